# baseline (speedup 1.0000x reference)
import numpy as np

import jax
import jax.numpy as jnp
from jax import lax
from jax.experimental import pallas as pl
from jax.experimental.pallas import tpu as pltpu
from jax.experimental.pallas import tpu_sc as plsc

NC = 2
NS = 16
NW = NC * NS
L = 16

B = 4096
D = 64
W32 = D // 2
HIST = 50
WISH = 20
HW = 72
TAGS_PAD = 8
B_PER_W = B // NW
DEPTH = 8

PERM = np.concatenate([np.r_[0:32:2], np.r_[1:32:2],
                       np.r_[32:64:2], np.r_[33:64:2]])


def _sc_pool_kernel(book_hbm, auth_hbm, lang_hbm, tag_hbm,
                    hw_idx_hbm, tag_idx_hbm, aid_hbm, lid_hbm,
                    u0_hbm, item_hbm,
                    hw_idx_v, tag_idx_v, aid_v, lid_v,
                    book_buf, tag_buf, a_rows, l_rows,
                    u0_st, it_st,
                    sem0, sem1, sem2, sem3, sem4, sem5, sem6, sem7):
    sems = (sem0, sem1, sem2, sem3, sem4, sem5, sem6, sem7)
    wid = lax.axis_index("s") * NC + lax.axis_index("c")
    base = wid * B_PER_W

    pltpu.sync_copy(hw_idx_hbm.at[pl.ds(base, B_PER_W)], hw_idx_v)
    pltpu.sync_copy(tag_idx_hbm.at[pl.ds(base, B_PER_W)], tag_idx_v)
    pltpu.sync_copy(aid_hbm.at[pl.ds(base, B_PER_W)], aid_v)
    pltpu.sync_copy(lid_hbm.at[pl.ds(base, B_PER_W)], lid_v)

    c_a = pltpu.async_copy(auth_hbm.at[aid_v], a_rows, sem0)
    c_l = pltpu.async_copy(lang_hbm.at[lid_v], l_rows, sem1)
    c_a.wait()
    c_l.wait()

    def hw_slice(r):
        return hw_idx_v.at[r]

    def tag_slice(r):
        return tag_idx_v.at[r]

    def issue(r, t):
        pltpu.async_copy(book_hbm.at[hw_slice(r)], book_buf.at[t], sems[t])
        pltpu.async_copy(tag_hbm.at[tag_slice(r)], tag_buf.at[t], sems[t])

    def wait_slot(r, t):
        pltpu.make_async_copy(book_hbm.at[hw_slice(r)], book_buf.at[t],
                              sems[t]).wait()
        pltpu.make_async_copy(tag_hbm.at[tag_slice(r)], tag_buf.at[t],
                              sems[t]).wait()

    for r0 in range(DEPTH - 1):
        issue(r0, r0)

    zero = jnp.zeros((L,), jnp.float32)

    def unpack2(v16):
        return plsc.unpack(plsc.bitcast(v16, jnp.bfloat16),
                           format=plsc.PackFormat.INTERLEAVED)

    def acc_rows(buf, t, j0, j1):
        acc = [zero] * 4
        for j in range(j0, j1):
            for h in range(2):
                a, b = unpack2(buf[t, j, pl.ds(16 * h, 16)])
                acc[2 * h] = acc[2 * h] + a
                acc[2 * h + 1] = acc[2 * h + 1] + b
        return acc

    def accum(r, t):
        uh = acc_rows(book_buf, t, 0, HIST)
        uw = acc_rows(book_buf, t, HIST, HIST + WISH)
        tg = acc_rows(tag_buf, t, 0, TAGS_PAD)
        be = []
        for h in range(2):
            a, b = unpack2(book_buf[t, HIST + WISH, pl.ds(16 * h, 16)])
            be.extend([a, b])
        al = []
        for h in range(2):
            a, b = unpack2(a_rows[r, pl.ds(16 * h, 16)])
            c, d = unpack2(l_rows[r, pl.ds(16 * h, 16)])
            al.extend([a + c, b + d])
        for c in range(4):
            sl = pl.ds(c * L, L)
            u0_st[r, sl] = uh[c] * (1.0 / 50.0) + uw[c] * (1.0 / 20.0)
            it_st[r, sl] = be[c] + al[c] + tg[c] * (1.0 / 5.0)

    def body(i, carry):
        for s in range(DEPTH):
            r = i * DEPTH + s
            wait_slot(r, s)
            accum(r, s)
            nxt = r + DEPTH - 1

            @pl.when(nxt < B_PER_W)
            def _():
                issue(nxt, (s + DEPTH - 1) % DEPTH)
        return carry

    lax.fori_loop(0, B_PER_W // DEPTH, body, 0)

    pltpu.sync_copy(u0_st, u0_hbm.at[pl.ds(base, B_PER_W)])
    pltpu.sync_copy(it_st, item_hbm.at[pl.ds(base, B_PER_W)])


def _sc_pool(book_p, auth_p, lang_p, tag_p, hw_idx, tag_idx, aid, lid):
    mesh = plsc.VectorSubcoreMesh(core_axis_name="c", subcore_axis_name="s")
    f32 = jnp.float32
    i32 = jnp.int32
    kern = pl.kernel(
        _sc_pool_kernel,
        out_type=(jax.ShapeDtypeStruct((B, D), f32),
                  jax.ShapeDtypeStruct((B, D), f32)),
        mesh=mesh,
        compiler_params=pltpu.CompilerParams(use_tc_tiling_on_sc=False,
                                             needs_layout_passes=False),
        scratch_types=(
            pltpu.VMEM((B_PER_W, HW), i32),
            pltpu.VMEM((B_PER_W, TAGS_PAD), i32),
            pltpu.VMEM((B_PER_W,), i32),
            pltpu.VMEM((B_PER_W,), i32),
            pltpu.VMEM((DEPTH, HW, W32), i32),
            pltpu.VMEM((DEPTH, TAGS_PAD, W32), i32),
            pltpu.VMEM((B_PER_W, W32), i32),
            pltpu.VMEM((B_PER_W, W32), i32),
            pltpu.VMEM((B_PER_W, D), f32),
            pltpu.VMEM((B_PER_W, D), f32),
            pltpu.SemaphoreType.DMA,
            pltpu.SemaphoreType.DMA,
            pltpu.SemaphoreType.DMA,
            pltpu.SemaphoreType.DMA,
            pltpu.SemaphoreType.DMA,
            pltpu.SemaphoreType.DMA,
            pltpu.SemaphoreType.DMA,
            pltpu.SemaphoreType.DMA,
        ),
    )
    return kern(book_p, auth_p, lang_p, tag_p, hw_idx, tag_idx, aid, lid)


def _tc_mlp_kernel(u0_ref, item_ref, dense_ref,
                   dw1_ref, db1_ref, dw2_ref, db2_ref,
                   uw1_ref, ub1_ref, uw2_ref, ub2_ref, uw3_ref, ub3_ref,
                   out_ref):
    f32 = jnp.float32
    u0 = u0_ref[...]
    h = jax.nn.relu(jnp.dot(u0, uw1_ref[...], preferred_element_type=f32)
                    + ub1_ref[...])
    h = jax.nn.relu(jnp.dot(h, uw2_ref[...], preferred_element_type=f32)
                    + ub2_ref[...])
    u_emb = jnp.dot(h, uw3_ref[...], preferred_element_type=f32) + ub3_ref[...]
    d = jax.nn.relu(jnp.dot(dense_ref[...], dw1_ref[...],
                            preferred_element_type=f32) + db1_ref[...])
    d_e = jnp.dot(d, dw2_ref[...], preferred_element_type=f32) + db2_ref[...]
    i_emb = item_ref[...] + d_e
    out_ref[...] = jnp.sum(u_emb * i_emb, axis=1, keepdims=True)


def _tc_mlp(u0, item_pool, dense8,
            dW1, db1, dW2, db2, uW1, ub1, uW2, ub2, uW3, ub3):
    f32 = jnp.float32
    BLK = 512
    grid = (B // BLK,)

    def batch_spec(cols):
        return pl.BlockSpec((BLK, cols), lambda i: (i, 0))

    def full_spec(a):
        return pl.BlockSpec(a.shape, lambda i: (0,) * a.ndim)

    return pl.pallas_call(
        _tc_mlp_kernel,
        grid=grid,
        in_specs=[
            batch_spec(D), batch_spec(D), batch_spec(8),
            full_spec(dW1), full_spec(db1), full_spec(dW2), full_spec(db2),
            full_spec(uW1), full_spec(ub1), full_spec(uW2), full_spec(ub2),
            full_spec(uW3), full_spec(ub3),
        ],
        out_specs=pl.BlockSpec((BLK, 1), lambda i: (i, 0)),
        out_shape=jax.ShapeDtypeStruct((B, 1), f32),
    )(u0, item_pool, dense8,
      dW1, db1, dW2, db2, uW1, ub1, uW2, ub2, uW3, ub3)


def _pack_table(t):
    bits = lax.bitcast_convert_type(t, jnp.int32)
    lo = lax.shift_right_logical(bits[:, 0::2] + 0x8000, 16)
    hi = (bits[:, 1::2] + 0x8000) & jnp.int32(np.int32(-65536))
    return lo | hi


def kernel(hist_ids, wish_ids, bid, auth, lang, tags, dense,
           book_emb, auth_emb, lang_emb, tag_emb,
           dense_W1, dense_b1, dense_W2, dense_b2,
           user_W1, user_b1, user_W2, user_b2, user_W3, user_b3):
    i32 = jnp.int32
    f32 = jnp.float32
    zcol = jnp.zeros((B, 1), i32)
    hw_idx = jnp.concatenate(
        [hist_ids.astype(i32), wish_ids.astype(i32),
         bid.astype(i32).reshape(B, 1), zcol], axis=1)
    tag_idx = jnp.pad(tags.astype(i32), ((0, 0), (0, TAGS_PAD - tags.shape[1])))

    u0, item_pool = _sc_pool(_pack_table(book_emb), _pack_table(auth_emb),
                             _pack_table(lang_emb), _pack_table(tag_emb),
                             hw_idx, tag_idx,
                             auth.astype(i32), lang.astype(i32))

    perm = jnp.asarray(PERM)
    dense8 = jnp.pad(dense.astype(f32), ((0, 0), (0, 8 - dense.shape[1])))
    dW1 = jnp.pad(dense_W1, ((0, 8 - dense_W1.shape[0]), (0, 0)))
    out = _tc_mlp(u0, item_pool, dense8,
                  dW1, dense_b1.reshape(1, -1),
                  dense_W2[:, perm], dense_b2[perm].reshape(1, -1),
                  user_W1[perm, :], user_b1.reshape(1, -1),
                  user_W2, user_b2.reshape(1, -1),
                  user_W3[:, perm], user_b3[perm].reshape(1, -1))
    return out


# speedup vs baseline: 4.2803x; 4.2803x over previous
import numpy as np

import jax
import jax.numpy as jnp
from jax import lax
from jax.experimental import pallas as pl
from jax.experimental.pallas import tpu as pltpu
from jax.experimental.pallas import tpu_sc as plsc

NC = 2
NS = 16
NW = NC * NS
L = 16

B = 4096
D = 64
HIST = 50
WISH = 20
HW = 72
TAGS_PAD = 8
B_PER_W = B // NW
DEPTH = 8

PERM = np.concatenate([np.r_[0:32:2], np.r_[1:32:2],
                       np.r_[32:64:2], np.r_[33:64:2]])


def _sc_pool_kernel(book_hbm, auth_hbm, lang_hbm, tag_hbm,
                    hw_idx_hbm, tag_idx_hbm, aid_hbm, lid_hbm,
                    u0_hbm, item_hbm,
                    hw_idx_v, tag_idx_v, aid_v, lid_v,
                    book_buf, tag_buf, a_rows, l_rows,
                    u0_st, it_st,
                    sem0, sem1, sem2, sem3, sem4, sem5, sem6, sem7):
    sems = (sem0, sem1, sem2, sem3, sem4, sem5, sem6, sem7)
    wid = lax.axis_index("s") * NC + lax.axis_index("c")
    base = wid * B_PER_W

    pltpu.sync_copy(hw_idx_hbm.at[pl.ds(base, B_PER_W)], hw_idx_v)
    pltpu.sync_copy(tag_idx_hbm.at[pl.ds(base, B_PER_W)], tag_idx_v)
    pltpu.sync_copy(aid_hbm.at[pl.ds(base, B_PER_W)], aid_v)
    pltpu.sync_copy(lid_hbm.at[pl.ds(base, B_PER_W)], lid_v)

    c_a = pltpu.async_copy(auth_hbm.at[aid_v], a_rows, sem0)
    c_l = pltpu.async_copy(lang_hbm.at[lid_v], l_rows, sem1)
    c_a.wait()
    c_l.wait()

    def issue(r, t):
        pltpu.async_copy(book_hbm.at[hw_idx_v.at[r]], book_buf.at[t], sems[t])
        pltpu.async_copy(tag_hbm.at[tag_idx_v.at[r]], tag_buf.at[t], sems[t])

    def wait_slot(r, t):
        pltpu.make_async_copy(book_hbm.at[hw_idx_v.at[r]], book_buf.at[t],
                              sems[t]).wait()
        pltpu.make_async_copy(tag_hbm.at[tag_idx_v.at[r]], tag_buf.at[t],
                              sems[t]).wait()

    for r0 in range(DEPTH - 1):
        issue(r0, r0)

    zero = jnp.zeros((L,), jnp.float32)
    unpack = lambda v: plsc.unpack(v, format=plsc.PackFormat.INTERLEAVED)

    def acc_rows(buf, t, j0, j1):
        acc = [zero] * 4
        for j in range(j0, j1):
            for h in range(2):
                a, b = unpack(buf[t, j, pl.ds(32 * h, 32)])
                acc[2 * h] = acc[2 * h] + a
                acc[2 * h + 1] = acc[2 * h + 1] + b
        return acc

    def accum(r, t):
        uh = acc_rows(book_buf, t, 0, HIST)
        uw = acc_rows(book_buf, t, HIST, HIST + WISH)
        tg = acc_rows(tag_buf, t, 0, TAGS_PAD)
        be = []
        for h in range(2):
            a, b = unpack(book_buf[t, HIST + WISH, pl.ds(32 * h, 32)])
            be.extend([a, b])
        al = []
        for h in range(2):
            a, b = unpack(a_rows[r, pl.ds(32 * h, 32)])
            c, d = unpack(l_rows[r, pl.ds(32 * h, 32)])
            al.extend([a + c, b + d])
        for c in range(4):
            sl = pl.ds(c * L, L)
            u0_st[r, sl] = uh[c] * (1.0 / 50.0) + uw[c] * (1.0 / 20.0)
            it_st[r, sl] = be[c] + al[c] + tg[c] * (1.0 / 5.0)

    def body(i, carry):
        for s in range(DEPTH):
            r = i * DEPTH + s
            wait_slot(r, s)
            accum(r, s)
            nxt = r + DEPTH - 1

            @pl.when(nxt < B_PER_W)
            def _():
                issue(nxt, (s + DEPTH - 1) % DEPTH)
        return carry

    lax.fori_loop(0, B_PER_W // DEPTH, body, 0)

    pltpu.sync_copy(u0_st, u0_hbm.at[pl.ds(base, B_PER_W)])
    pltpu.sync_copy(it_st, item_hbm.at[pl.ds(base, B_PER_W)])


def _sc_pool(book_emb, auth_emb, lang_emb, tag_emb, hw_idx, tag_idx, aid, lid):
    mesh = plsc.VectorSubcoreMesh(core_axis_name="c", subcore_axis_name="s")
    f32 = jnp.float32
    bf16 = jnp.bfloat16
    kern = pl.kernel(
        _sc_pool_kernel,
        out_type=(jax.ShapeDtypeStruct((B, D), f32),
                  jax.ShapeDtypeStruct((B, D), f32)),
        mesh=mesh,
        compiler_params=pltpu.CompilerParams(use_tc_tiling_on_sc=False,
                                             needs_layout_passes=False),
        scratch_types=(
            pltpu.VMEM((B_PER_W, HW), jnp.int32),
            pltpu.VMEM((B_PER_W, TAGS_PAD), jnp.int32),
            pltpu.VMEM((B_PER_W,), jnp.int32),
            pltpu.VMEM((B_PER_W,), jnp.int32),
            pltpu.VMEM((DEPTH, HW, D), bf16),
            pltpu.VMEM((DEPTH, TAGS_PAD, D), bf16),
            pltpu.VMEM((B_PER_W, D), bf16),
            pltpu.VMEM((B_PER_W, D), bf16),
            pltpu.VMEM((B_PER_W, D), f32),
            pltpu.VMEM((B_PER_W, D), f32),
            pltpu.SemaphoreType.DMA,
            pltpu.SemaphoreType.DMA,
            pltpu.SemaphoreType.DMA,
            pltpu.SemaphoreType.DMA,
            pltpu.SemaphoreType.DMA,
            pltpu.SemaphoreType.DMA,
            pltpu.SemaphoreType.DMA,
            pltpu.SemaphoreType.DMA,
        ),
    )
    return kern(book_emb, auth_emb, lang_emb, tag_emb,
                hw_idx, tag_idx, aid, lid)


def _tc_mlp_kernel(u0_ref, item_ref, dense_ref,
                   dw1_ref, db1_ref, dw2_ref, db2_ref,
                   uw1_ref, ub1_ref, uw2_ref, ub2_ref, uw3_ref, ub3_ref,
                   out_ref):
    f32 = jnp.float32
    u0 = u0_ref[...]
    h = jax.nn.relu(jnp.dot(u0, uw1_ref[...], preferred_element_type=f32)
                    + ub1_ref[...])
    h = jax.nn.relu(jnp.dot(h, uw2_ref[...], preferred_element_type=f32)
                    + ub2_ref[...])
    u_emb = jnp.dot(h, uw3_ref[...], preferred_element_type=f32) + ub3_ref[...]
    d = jax.nn.relu(jnp.dot(dense_ref[...], dw1_ref[...],
                            preferred_element_type=f32) + db1_ref[...])
    d_e = jnp.dot(d, dw2_ref[...], preferred_element_type=f32) + db2_ref[...]
    i_emb = item_ref[...] + d_e
    out_ref[...] = jnp.sum(u_emb * i_emb, axis=1, keepdims=True)


def _tc_mlp(u0, item_pool, dense8,
            dW1, db1, dW2, db2, uW1, ub1, uW2, ub2, uW3, ub3):
    f32 = jnp.float32
    BLK = 512
    grid = (B // BLK,)

    def batch_spec(cols):
        return pl.BlockSpec((BLK, cols), lambda i: (i, 0))

    def full_spec(a):
        return pl.BlockSpec(a.shape, lambda i: (0,) * a.ndim)

    return pl.pallas_call(
        _tc_mlp_kernel,
        grid=grid,
        in_specs=[
            batch_spec(D), batch_spec(D), batch_spec(8),
            full_spec(dW1), full_spec(db1), full_spec(dW2), full_spec(db2),
            full_spec(uW1), full_spec(ub1), full_spec(uW2), full_spec(ub2),
            full_spec(uW3), full_spec(ub3),
        ],
        out_specs=pl.BlockSpec((BLK, 1), lambda i: (i, 0)),
        out_shape=jax.ShapeDtypeStruct((B, 1), f32),
    )(u0, item_pool, dense8,
      dW1, db1, dW2, db2, uW1, ub1, uW2, ub2, uW3, ub3)


def kernel(hist_ids, wish_ids, bid, auth, lang, tags, dense,
           book_emb, auth_emb, lang_emb, tag_emb,
           dense_W1, dense_b1, dense_W2, dense_b2,
           user_W1, user_b1, user_W2, user_b2, user_W3, user_b3):
    i32 = jnp.int32
    f32 = jnp.float32
    bf16 = jnp.bfloat16
    zcol = jnp.zeros((B, 1), i32)
    hw_idx = jnp.concatenate(
        [hist_ids.astype(i32), wish_ids.astype(i32),
         bid.astype(i32).reshape(B, 1), zcol], axis=1)
    tag_idx = jnp.pad(tags.astype(i32), ((0, 0), (0, TAGS_PAD - tags.shape[1])))

    u0, item_pool = _sc_pool(book_emb.astype(bf16), auth_emb.astype(bf16),
                             lang_emb.astype(bf16), tag_emb.astype(bf16),
                             hw_idx, tag_idx,
                             auth.astype(i32), lang.astype(i32))

    perm = jnp.asarray(PERM)
    dense8 = jnp.pad(dense.astype(f32), ((0, 0), (0, 8 - dense.shape[1])))
    dW1 = jnp.pad(dense_W1, ((0, 8 - dense_W1.shape[0]), (0, 0)))
    out = _tc_mlp(u0, item_pool, dense8,
                  dW1, dense_b1.reshape(1, -1),
                  dense_W2[:, perm], dense_b2[perm].reshape(1, -1),
                  user_W1[perm, :], user_b1.reshape(1, -1),
                  user_W2, user_b2.reshape(1, -1),
                  user_W3[:, perm], user_b3[perm].reshape(1, -1))
    return out


# speedup vs baseline: 5.6638x; 1.3232x over previous
import numpy as np

import jax
import jax.numpy as jnp
from jax import lax
from jax.experimental import pallas as pl
from jax.experimental.pallas import tpu as pltpu
from jax.experimental.pallas import tpu_sc as plsc

NC = 2
NS = 16
NW = NC * NS
L = 16

B = 4096
D = 64
HIST = 50
WISH = 20
HW = 72
TAGS_PAD = 16
B_PER_W = B // NW
DEPTH = 8

PERM = np.concatenate([np.r_[0:32:2], np.r_[1:32:2],
                       np.r_[32:64:2], np.r_[33:64:2]])


def _sc_pool_kernel(book_hbm, auth_hbm, lang_hbm, tag_hbm,
                    hw_idx_hbm, tag_idx_hbm, aid_hbm, lid_hbm,
                    u0_hbm, item_hbm,
                    hw_idx_v, tag_idx_v, aid_v, lid_v,
                    book_buf, tag_tbl, a_rows, l_rows,
                    u0_st, it_st,
                    sem0, sem1, sem2, sem3, sem4, sem5, sem6, sem7):
    sems = (sem0, sem1, sem2, sem3, sem4, sem5, sem6, sem7)
    wid = lax.axis_index("s") * NC + lax.axis_index("c")
    base = wid * B_PER_W

    pltpu.sync_copy(hw_idx_hbm.at[pl.ds(base, B_PER_W)], hw_idx_v)
    pltpu.sync_copy(tag_idx_hbm.at[pl.ds(base, B_PER_W)], tag_idx_v)
    pltpu.sync_copy(aid_hbm.at[pl.ds(base, B_PER_W)], aid_v)
    pltpu.sync_copy(lid_hbm.at[pl.ds(base, B_PER_W)], lid_v)
    pltpu.sync_copy(tag_hbm, tag_tbl)

    c_a = pltpu.async_copy(auth_hbm.at[aid_v], a_rows, sem0)
    c_l = pltpu.async_copy(lang_hbm.at[lid_v], l_rows, sem1)
    c_a.wait()
    c_l.wait()

    def issue(r, t):
        pltpu.async_copy(book_hbm.at[hw_idx_v.at[r]], book_buf.at[t], sems[t])

    def wait_slot(r, t):
        pltpu.make_async_copy(book_hbm.at[hw_idx_v.at[r]], book_buf.at[t],
                              sems[t]).wait()

    for r0 in range(DEPTH - 1):
        issue(r0, r0)

    zero = jnp.zeros((L,), jnp.float32)
    unpack = lambda v: plsc.unpack(v, format=plsc.PackFormat.INTERLEAVED)

    def acc_rows(buf, t, j0, j1):
        acc = [zero] * 4
        for j in range(j0, j1):
            for h in range(2):
                a, b = unpack(buf[t, j, pl.ds(32 * h, 32)])
                acc[2 * h] = acc[2 * h] + a
                acc[2 * h + 1] = acc[2 * h + 1] + b
        return acc

    col16 = jnp.arange(16, dtype=jnp.int32)

    def accum(r, t):
        uh = acc_rows(book_buf, t, 0, HIST)
        uw = acc_rows(book_buf, t, HIST, HIST + WISH)
        tg = [zero] * 4
        tag_row_ids = tag_idx_v[r, pl.ds(0, 16)]
        for j in range(5):
            rows16 = jnp.full((16,), tag_row_ids[j], jnp.int32)
            for h in range(2):
                w = plsc.load_gather(tag_tbl, [rows16, col16 + (16 * h)])
                a, b = unpack(plsc.bitcast(w, jnp.bfloat16))
                tg[2 * h] = tg[2 * h] + a
                tg[2 * h + 1] = tg[2 * h + 1] + b
        be = []
        for h in range(2):
            a, b = unpack(book_buf[t, HIST + WISH, pl.ds(32 * h, 32)])
            be.extend([a, b])
        al = []
        for h in range(2):
            a, b = unpack(a_rows[r, pl.ds(32 * h, 32)])
            c, d = unpack(l_rows[r, pl.ds(32 * h, 32)])
            al.extend([a + c, b + d])
        for c in range(4):
            sl = pl.ds(c * L, L)
            u0_st[r, sl] = uh[c] * (1.0 / 50.0) + uw[c] * (1.0 / 20.0)
            it_st[r, sl] = be[c] + al[c] + tg[c] * (1.0 / 5.0)

    def body(i, carry):
        for s in range(DEPTH):
            r = i * DEPTH + s
            wait_slot(r, s)
            accum(r, s)
            nxt = r + DEPTH - 1

            @pl.when(nxt < B_PER_W)
            def _():
                issue(nxt, (s + DEPTH - 1) % DEPTH)
        return carry

    lax.fori_loop(0, B_PER_W // DEPTH, body, 0)

    pltpu.sync_copy(u0_st, u0_hbm.at[pl.ds(base, B_PER_W)])
    pltpu.sync_copy(it_st, item_hbm.at[pl.ds(base, B_PER_W)])


def _sc_pool(book_emb, auth_emb, lang_emb, tag_p, hw_idx, tag_idx, aid, lid):
    mesh = plsc.VectorSubcoreMesh(core_axis_name="c", subcore_axis_name="s")
    f32 = jnp.float32
    bf16 = jnp.bfloat16
    kern = pl.kernel(
        _sc_pool_kernel,
        out_type=(jax.ShapeDtypeStruct((B, D), f32),
                  jax.ShapeDtypeStruct((B, D), f32)),
        mesh=mesh,
        compiler_params=pltpu.CompilerParams(use_tc_tiling_on_sc=False,
                                             needs_layout_passes=False),
        scratch_types=(
            pltpu.VMEM((B_PER_W, HW), jnp.int32),
            pltpu.VMEM((B_PER_W, TAGS_PAD), jnp.int32),
            pltpu.VMEM((B_PER_W,), jnp.int32),
            pltpu.VMEM((B_PER_W,), jnp.int32),
            pltpu.VMEM((DEPTH, HW, D), bf16),
            pltpu.VMEM(tag_p.shape, jnp.int32),
            pltpu.VMEM((B_PER_W, D), bf16),
            pltpu.VMEM((B_PER_W, D), bf16),
            pltpu.VMEM((B_PER_W, D), f32),
            pltpu.VMEM((B_PER_W, D), f32),
            pltpu.SemaphoreType.DMA,
            pltpu.SemaphoreType.DMA,
            pltpu.SemaphoreType.DMA,
            pltpu.SemaphoreType.DMA,
            pltpu.SemaphoreType.DMA,
            pltpu.SemaphoreType.DMA,
            pltpu.SemaphoreType.DMA,
            pltpu.SemaphoreType.DMA,
        ),
    )
    return kern(book_emb, auth_emb, lang_emb, tag_p,
                hw_idx, tag_idx, aid, lid)


def _tc_mlp_kernel(u0_ref, item_ref, dense_ref,
                   dw1_ref, db1_ref, dw2_ref, db2_ref,
                   uw1_ref, ub1_ref, uw2_ref, ub2_ref, uw3_ref, ub3_ref,
                   out_ref):
    f32 = jnp.float32
    u0 = u0_ref[...]
    h = jax.nn.relu(jnp.dot(u0, uw1_ref[...], preferred_element_type=f32)
                    + ub1_ref[...])
    h = jax.nn.relu(jnp.dot(h, uw2_ref[...], preferred_element_type=f32)
                    + ub2_ref[...])
    u_emb = jnp.dot(h, uw3_ref[...], preferred_element_type=f32) + ub3_ref[...]
    d = jax.nn.relu(jnp.dot(dense_ref[...], dw1_ref[...],
                            preferred_element_type=f32) + db1_ref[...])
    d_e = jnp.dot(d, dw2_ref[...], preferred_element_type=f32) + db2_ref[...]
    i_emb = item_ref[...] + d_e
    out_ref[...] = jnp.sum(u_emb * i_emb, axis=1, keepdims=True)


def _tc_mlp(u0, item_pool, dense8,
            dW1, db1, dW2, db2, uW1, ub1, uW2, ub2, uW3, ub3):
    f32 = jnp.float32
    BLK = 512
    grid = (B // BLK,)

    def batch_spec(cols):
        return pl.BlockSpec((BLK, cols), lambda i: (i, 0))

    def full_spec(a):
        return pl.BlockSpec(a.shape, lambda i: (0,) * a.ndim)

    return pl.pallas_call(
        _tc_mlp_kernel,
        grid=grid,
        in_specs=[
            batch_spec(D), batch_spec(D), batch_spec(8),
            full_spec(dW1), full_spec(db1), full_spec(dW2), full_spec(db2),
            full_spec(uW1), full_spec(ub1), full_spec(uW2), full_spec(ub2),
            full_spec(uW3), full_spec(ub3),
        ],
        out_specs=pl.BlockSpec((BLK, 1), lambda i: (i, 0)),
        out_shape=jax.ShapeDtypeStruct((B, 1), f32),
    )(u0, item_pool, dense8,
      dW1, db1, dW2, db2, uW1, ub1, uW2, ub2, uW3, ub3)


def kernel(hist_ids, wish_ids, bid, auth, lang, tags, dense,
           book_emb, auth_emb, lang_emb, tag_emb,
           dense_W1, dense_b1, dense_W2, dense_b2,
           user_W1, user_b1, user_W2, user_b2, user_W3, user_b3):
    i32 = jnp.int32
    f32 = jnp.float32
    bf16 = jnp.bfloat16
    zcol = jnp.zeros((B, 1), i32)
    hw_idx = jnp.concatenate(
        [hist_ids.astype(i32), wish_ids.astype(i32),
         bid.astype(i32).reshape(B, 1), zcol], axis=1)
    tag_idx = jnp.pad(tags.astype(i32), ((0, 0), (0, TAGS_PAD - tags.shape[1])))

    tag_p = lax.bitcast_convert_type(
        tag_emb.astype(bf16).reshape(tag_emb.shape[0], D // 2, 2), i32)
    u0, item_pool = _sc_pool(book_emb.astype(bf16), auth_emb.astype(bf16),
                             lang_emb.astype(bf16), tag_p,
                             hw_idx, tag_idx,
                             auth.astype(i32), lang.astype(i32))

    perm = jnp.asarray(PERM)
    dense8 = jnp.pad(dense.astype(f32), ((0, 0), (0, 8 - dense.shape[1])))
    dW1 = jnp.pad(dense_W1, ((0, 8 - dense_W1.shape[0]), (0, 0)))
    out = _tc_mlp(u0, item_pool, dense8,
                  dW1, dense_b1.reshape(1, -1),
                  dense_W2[:, perm], dense_b2[perm].reshape(1, -1),
                  user_W1[perm, :], user_b1.reshape(1, -1),
                  user_W2, user_b2.reshape(1, -1),
                  user_W3[:, perm], user_b3[perm].reshape(1, -1))
    return out
